# Initial kernel scaffold; baseline (speedup 1.0000x reference)
import jax
import jax.numpy as jnp
from jax.experimental import pallas as pl


def kernel(input0, input1):
    raise NotImplementedError("write your pallas kernel here")


# speedup vs baseline: 1.5738x; 1.5738x over previous
import functools
import jax
import jax.numpy as jnp
from jax import lax
from jax.experimental import pallas as pl
from jax.experimental.pallas import tpu as pltpu, tpu_sc as plsc

_D = 64
_B = 16384 * 50
_CH = 128

_info = plsc.get_sparse_core_info()
_NC, _NS = _info.num_cores, _info.num_subcores
_NW = _NC * _NS
_BPW = _B // _NW
_NCHUNK = _BPW // _CH


def _gather_body(table_hbm, idx_hbm, out_hbm, idx_v, rows_v, sem_g):
    wid = lax.axis_index("s") * _NC + lax.axis_index("c")
    base = wid * _BPW

    def body(c, _):
        off = base + c * _CH
        pltpu.sync_copy(idx_hbm.at[pl.ds(off, _CH)], idx_v)
        pltpu.async_copy(table_hbm.at[idx_v], rows_v, sem_g).wait()
        pltpu.sync_copy(rows_v, out_hbm.at[pl.ds(off, _CH)])
        return _

    lax.fori_loop(0, _NCHUNK, body, None)


_gather_call = pl.kernel(
    _gather_body,
    out_type=jax.ShapeDtypeStruct((_B, _D), jnp.float32),
    mesh=plsc.VectorSubcoreMesh(core_axis_name="c", subcore_axis_name="s"),
    scratch_types=[
        pltpu.VMEM((_CH,), jnp.int32),
        pltpu.VMEM((_CH, _D), jnp.float32),
        pltpu.SemaphoreType.DMA,
    ],
    compiler_params=pltpu.CompilerParams(use_tc_tiling_on_sc=False),
)


def kernel(input0, input1):
    idx = input1.reshape(-1).astype(jnp.int32)
    out = _gather_call(input0, idx)
    return out.reshape(input1.shape + (_D,))


# speedup vs baseline: 1.8709x; 1.1888x over previous
import functools
import jax
import jax.numpy as jnp
from jax import lax
from jax.experimental import pallas as pl
from jax.experimental.pallas import tpu as pltpu, tpu_sc as plsc

_D = 64
_B = 16384 * 50
_KS = 4
_CH = _KS * 128
_info = plsc.get_sparse_core_info()
_NC, _NS = _info.num_cores, _info.num_subcores
_NW = _NC * _NS
_BPW = _B // _NW
_NCH = _BPW // _CH
_NL = _NCH // 2


def _gather_body(table_hbm, idx_hbm, out_hbm,
                 idx_v, rows_v, si0, si1, sg0, sg1, ss0, ss1):
    wid = lax.axis_index("s") * _NC + lax.axis_index("c")
    base = wid * _BPW
    bblk = wid * (_BPW // 128)

    sem_i = (si0, si1)
    sem_g = (sg0, sg1)
    sem_s = (ss0, ss1)

    def start_idx(c, b):
        pltpu.async_copy(idx_hbm.at[pl.ds(bblk + c * _KS, _KS)],
                         idx_v.at[b], sem_i[b])

    def wait_idx(b):
        pltpu.make_async_copy(idx_hbm.at[pl.ds(0, _KS)],
                              idx_v.at[b], sem_i[b]).wait()

    def start_gather(b):
        for j in range(_KS):
            pltpu.async_copy(table_hbm.at[idx_v.at[b, j]],
                             rows_v.at[b, pl.ds(j * 128, 128)], sem_g[b])

    def wait_gather(b):
        pltpu.make_async_copy(table_hbm.at[pl.ds(0, _CH)],
                              rows_v.at[b], sem_g[b]).wait()

    def start_store(c, b):
        pltpu.async_copy(rows_v.at[b], out_hbm.at[pl.ds(base + c * _CH, _CH)],
                         sem_s[b])

    def wait_store(b):
        pltpu.make_async_copy(rows_v.at[b], out_hbm.at[pl.ds(0, _CH)],
                              sem_s[b]).wait()

    start_idx(0, 0)
    start_idx(1, 1)

    def body(g, _):
        for b in range(2):
            c = 2 * g + b

            @pl.when(g > 0)
            def _():
                wait_store(b)

            wait_idx(b)
            start_gather(b)
            wait_gather(b)
            start_store(c, b)

            @pl.when(g < _NL - 1)
            def _():
                start_idx(c + 2, b)

        return _

    lax.fori_loop(0, _NL, body, None)
    wait_store(0)
    wait_store(1)


_gather_call = pl.kernel(
    _gather_body,
    out_type=jax.ShapeDtypeStruct((_B, _D), jnp.float32),
    mesh=plsc.VectorSubcoreMesh(core_axis_name="c", subcore_axis_name="s"),
    scratch_types=[
        pltpu.VMEM((2, _KS, 128), jnp.int32),
        pltpu.VMEM((2, _CH, _D), jnp.float32),
        pltpu.SemaphoreType.DMA,
        pltpu.SemaphoreType.DMA,
        pltpu.SemaphoreType.DMA,
        pltpu.SemaphoreType.DMA,
        pltpu.SemaphoreType.DMA,
        pltpu.SemaphoreType.DMA,
    ],
    compiler_params=pltpu.CompilerParams(use_tc_tiling_on_sc=False),
)


def kernel(input0, input1):
    idx = input1.reshape(-1, 128).astype(jnp.int32)
    out = _gather_call(input0, idx)
    return out.reshape(input1.shape + (_D,))
